# baseline (speedup 1.0000x reference)
import functools

import jax
import jax.numpy as jnp
from jax import lax
from jax.experimental import pallas as pl
from jax.experimental.pallas import tpu as pltpu
from jax.experimental.pallas import tpu_sc as plsc

N = 10000
E = 320000
D = 128
NC = 2
NS = 16
NW = NC * NS
EW = E // NW
C = 80
NCHUNK = EW // C
RPT = 624
TAIL = N - RPT * NS
G = D // 16


def _sc_segment_sum(interpret=False):
    mesh = plsc.VectorSubcoreMesh(core_axis_name="c", subcore_axis_name="s",
                                  num_cores=NC, num_subcores=NS)

    @functools.partial(
        pl.kernel,
        out_type=jax.ShapeDtypeStruct((NC, N, D), jnp.float32),
        mesh=mesh,
        scratch_types=(
            [pltpu.VMEM_SHARED((N, D), jnp.float32)]
            + [pltpu.VMEM((8, C), jnp.int32)]
            + [pltpu.VMEM((8, C), jnp.int32)]
            + [pltpu.VMEM((8, C), jnp.float32)]
            + [pltpu.VMEM((C, D), jnp.float32)] * 4
            + [pltpu.SemaphoreType.DMA] * 16
        ),
        interpret=interpret,
    )
    def seg_sum(x_hbm, src_hbm, dst_hbm, w_hbm, out_hbm,
                acc, srcb, dstb, wb, b0, b1, b2, b3, *sems):
        cid = lax.axis_index("c")
        sid = lax.axis_index("s")
        wid = cid * NS + sid

        bufs = (b0, b1, b2, b3)
        gsems, ssems, isems = sems[0:4], sems[4:8], sems[8:16]
        g0 = b0

        zero = jnp.zeros((16,), jnp.float32)

        def zbody(i, _):
            for g in range(G):
                g0[i, pl.ds(g * 16, 16)] = zero
            return 0

        lax.fori_loop(0, C, zbody, 0, unroll=False)
        base = sid * RPT
        nfull = RPT // C
        rem = RPT - nfull * C
        for k in range(nfull):
            pltpu.sync_copy(g0, acc.at[pl.ds(base + k * C, C)])
        pltpu.sync_copy(g0.at[pl.ds(0, rem)],
                        acc.at[pl.ds(base + nfull * C, rem)])

        @pl.when(sid == NS - 1)
        def _zero_tail():
            pltpu.sync_copy(g0.at[pl.ds(0, TAIL)],
                            acc.at[pl.ds(NS * RPT, TAIL)])

        plsc.subcore_barrier()

        e_base = wid * EW

        def idx_copies(j, slot):
            sem = isems[slot]
            off = e_base + j * C
            yield pltpu.make_async_copy(
                src_hbm.at[pl.ds(off, C)], srcb.at[slot], sem)
            yield pltpu.make_async_copy(
                dst_hbm.at[pl.ds(off, C)], dstb.at[slot], sem)
            yield pltpu.make_async_copy(
                w_hbm.at[pl.ds(off, C)], wb.at[slot], sem)

        def issue_idx(j, slot):
            for cpy in idx_copies(j, slot):
                cpy.start()

        def wait_idx(j, slot):
            for cpy in idx_copies(j, slot):
                cpy.wait()

        def gather_copy(islot):
            bslot = islot % 4
            return pltpu.make_async_copy(x_hbm.at[srcb.at[islot]],
                                         bufs[bslot], gsems[bslot])

        def scatter_copy(islot):
            bslot = islot % 4
            return pltpu.make_async_copy(bufs[bslot], acc.at[dstb.at[islot]],
                                         ssems[bslot])

        def scale(islot):
            buf = bufs[islot % 4]

            def e_body(t, _):
                wv = wb[islot, pl.ds(t * 16, 16)]
                for k in range(16):
                    c = t * 16 + k
                    ws = wv[k]
                    for g in range(G):
                        sl = pl.ds(g * 16, 16)
                        buf[c, sl] = buf[c, sl] * ws
                return 0

            lax.fori_loop(0, C // 16, e_body, 0, unroll=False)

        for j0 in range(4):
            issue_idx(j0, j0)
        wait_idx(0, 0)
        gather_copy(0).start()
        wait_idx(1, 1)
        gather_copy(1).start()

        def oct_body(k, _):
            for q in range(8):
                j = 8 * k + q

                if q >= 2:
                    scatter_copy((q - 2) % 8).wait()
                else:
                    @pl.when(k > 0)
                    def _wait_prev_scatter():
                        scatter_copy((q - 2) % 8).wait()

                issue_idx(j + 4, (q + 4) % 8)
                wait_idx(j + 2, (q + 2) % 8)
                gather_copy((q + 2) % 8).start()
                gather_copy(q).wait()
                scale(q)
                scatter_copy(q).start(add=True)
            return 0

        NLOOP = (NCHUNK - 5) // 8
        lax.fori_loop(0, NLOOP, oct_body, 0, unroll=False)

        for j in range(NLOOP * 8, NCHUNK):
            q = j % 8
            scatter_copy((q - 2) % 8).wait()
            if j + 4 < NCHUNK:
                issue_idx(j + 4, (q + 4) % 8)
            if j + 2 < NCHUNK:
                wait_idx(j + 2, (q + 2) % 8)
                gather_copy((q + 2) % 8).start()
            gather_copy(q).wait()
            scale(q)
            scatter_copy(q).start(add=True)
        scatter_copy((NCHUNK - 2) % 8).wait()
        scatter_copy((NCHUNK - 1) % 8).wait()

        plsc.subcore_barrier()
        pl.delay(1000)
        pltpu.sync_copy(acc.at[pl.ds(base, RPT)],
                        out_hbm.at[cid, pl.ds(base, RPT)])

        @pl.when(sid == NS - 1)
        def _copy_tail():
            pltpu.sync_copy(acc.at[pl.ds(NS * RPT, TAIL)],
                            out_hbm.at[cid, pl.ds(NS * RPT, TAIL)])

    return seg_sum


def _tc_body(p_ref, w_ref, b_ref, o_ref):
    s = p_ref[0] + p_ref[1]
    o_ref[...] = jnp.dot(s, w_ref[...],
                         preferred_element_type=jnp.float32) + b_ref[...]


def _tc_project(partial, weight, bias2d, interpret=False):
    BN = 1000
    grid = (N // BN,)
    return pl.pallas_call(
        _tc_body,
        out_shape=jax.ShapeDtypeStruct((N, D), jnp.float32),
        grid=grid,
        in_specs=[
            pl.BlockSpec((NC, BN, D), lambda i: (0, i, 0)),
            pl.BlockSpec((D, D), lambda i: (0, 0)),
            pl.BlockSpec((1, D), lambda i: (0, 0)),
        ],
        out_specs=pl.BlockSpec((BN, D), lambda i: (i, 0)),
        interpret=interpret,
    )(partial, weight, bias2d)


def kernel(x, edge_index, edge_weight, weight, bias):
    src = edge_index[1].astype(jnp.int32)
    dst = edge_index[0].astype(jnp.int32)
    partial = _sc_segment_sum()(x, src, dst, edge_weight)
    return _tc_project(partial, weight, bias.reshape(1, D))


# speedup vs baseline: 1.1533x; 1.1533x over previous
import functools

import jax
import jax.numpy as jnp
from jax import lax
from jax.experimental import pallas as pl
from jax.experimental.pallas import tpu as pltpu
from jax.experimental.pallas import tpu_sc as plsc

N = 10000
E = 320000
D = 128
NC = 2
NS = 16
NW = NC * NS
EW = E // NW
C = 80
NCHUNK = EW // C
RPT = 624
TAIL = N - RPT * NS
G = D // 16


def _sc_segment_sum(interpret=False):
    mesh = plsc.VectorSubcoreMesh(core_axis_name="c", subcore_axis_name="s",
                                  num_cores=NC, num_subcores=NS)

    @functools.partial(
        pl.kernel,
        out_type=jax.ShapeDtypeStruct((NC, N, D), jnp.float32),
        mesh=mesh,
        scratch_types=(
            [pltpu.VMEM_SHARED((N, D), jnp.float32)]
            + [pltpu.VMEM((8, C), jnp.int32)]
            + [pltpu.VMEM((8, C), jnp.int32)]
            + [pltpu.VMEM((8, C), jnp.float32)]
            + [pltpu.VMEM((C, D), jnp.float32)] * 4
            + [pltpu.SemaphoreType.DMA] * 16
        ),
        interpret=interpret,
    )
    def seg_sum(x_hbm, src_hbm, dst_hbm, w_hbm, out_hbm,
                acc, srcb, dstb, wb, b0, b1, b2, b3, *sems):
        cid = lax.axis_index("c")
        sid = lax.axis_index("s")
        wid = cid * NS + sid

        bufs = (b0, b1, b2, b3)
        gsems, ssems, isems = sems[0:4], sems[4:8], sems[8:16]
        g0 = b0

        zero = jnp.zeros((16,), jnp.float32)

        def zbody(i, _):
            for g in range(G):
                g0[i, pl.ds(g * 16, 16)] = zero
            return 0

        lax.fori_loop(0, C, zbody, 0, unroll=False)
        base = sid * RPT
        nfull = RPT // C
        rem = RPT - nfull * C
        for k in range(nfull):
            pltpu.sync_copy(g0, acc.at[pl.ds(base + k * C, C)])
        pltpu.sync_copy(g0.at[pl.ds(0, rem)],
                        acc.at[pl.ds(base + nfull * C, rem)])

        @pl.when(sid == NS - 1)
        def _zero_tail():
            pltpu.sync_copy(g0.at[pl.ds(0, TAIL)],
                            acc.at[pl.ds(NS * RPT, TAIL)])

        plsc.subcore_barrier()

        e_base = wid * EW

        def idx_copies(j, slot):
            sem = isems[slot]
            off = e_base + j * C
            yield pltpu.make_async_copy(
                src_hbm.at[pl.ds(off, C)], srcb.at[slot], sem)
            yield pltpu.make_async_copy(
                dst_hbm.at[pl.ds(off, C)], dstb.at[slot], sem)
            yield pltpu.make_async_copy(
                w_hbm.at[pl.ds(off, C)], wb.at[slot], sem)

        def issue_idx(j, slot):
            for cpy in idx_copies(j, slot):
                cpy.start()

        def wait_idx(j, slot):
            for cpy in idx_copies(j, slot):
                cpy.wait()

        def gather_copy(islot):
            bslot = islot % 4
            return pltpu.make_async_copy(x_hbm.at[srcb.at[islot]],
                                         bufs[bslot], gsems[bslot])

        def scatter_copy(islot):
            bslot = islot % 4
            return pltpu.make_async_copy(bufs[bslot], acc.at[dstb.at[islot]],
                                         ssems[bslot])

        def scale(islot):
            buf = bufs[islot % 4]

            @functools.partial(plsc.parallel_loop, 0, C // 16)
            def e_body(t):
                wv = wb[islot, pl.ds(t * 16, 16)]
                for k in range(16):
                    c = t * 16 + k
                    ws = wv[k]
                    for g in range(G):
                        sl = pl.ds(g * 16, 16)
                        buf[c, sl] = buf[c, sl] * ws

        for j0 in range(4):
            issue_idx(j0, j0)
        wait_idx(0, 0)
        gather_copy(0).start()
        wait_idx(1, 1)
        gather_copy(1).start()

        def oct_body(k, _):
            for q in range(8):
                j = 8 * k + q

                if q >= 2:
                    scatter_copy((q - 2) % 8).wait()
                else:
                    @pl.when(k > 0)
                    def _wait_prev_scatter():
                        scatter_copy((q - 2) % 8).wait()

                issue_idx(j + 4, (q + 4) % 8)
                wait_idx(j + 2, (q + 2) % 8)
                gather_copy((q + 2) % 8).start()
                gather_copy(q).wait()
                scale(q)
                scatter_copy(q).start(add=True)
            return 0

        NLOOP = (NCHUNK - 5) // 8
        lax.fori_loop(0, NLOOP, oct_body, 0, unroll=False)

        for j in range(NLOOP * 8, NCHUNK):
            q = j % 8
            scatter_copy((q - 2) % 8).wait()
            if j + 4 < NCHUNK:
                issue_idx(j + 4, (q + 4) % 8)
            if j + 2 < NCHUNK:
                wait_idx(j + 2, (q + 2) % 8)
                gather_copy((q + 2) % 8).start()
            gather_copy(q).wait()
            scale(q)
            scatter_copy(q).start(add=True)
        scatter_copy((NCHUNK - 2) % 8).wait()
        scatter_copy((NCHUNK - 1) % 8).wait()

        plsc.subcore_barrier()
        pl.delay(1000)
        pltpu.sync_copy(acc.at[pl.ds(base, RPT)],
                        out_hbm.at[cid, pl.ds(base, RPT)])

        @pl.when(sid == NS - 1)
        def _copy_tail():
            pltpu.sync_copy(acc.at[pl.ds(NS * RPT, TAIL)],
                            out_hbm.at[cid, pl.ds(NS * RPT, TAIL)])

    return seg_sum


def _tc_body(p_ref, w_ref, b_ref, o_ref):
    s = p_ref[0] + p_ref[1]
    o_ref[...] = jnp.dot(s, w_ref[...],
                         preferred_element_type=jnp.float32) + b_ref[...]


def _tc_project(partial, weight, bias2d, interpret=False):
    BN = 1000
    grid = (N // BN,)
    return pl.pallas_call(
        _tc_body,
        out_shape=jax.ShapeDtypeStruct((N, D), jnp.float32),
        grid=grid,
        in_specs=[
            pl.BlockSpec((NC, BN, D), lambda i: (0, i, 0)),
            pl.BlockSpec((D, D), lambda i: (0, 0)),
            pl.BlockSpec((1, D), lambda i: (0, 0)),
        ],
        out_specs=pl.BlockSpec((BN, D), lambda i: (i, 0)),
        interpret=interpret,
    )(partial, weight, bias2d)


def kernel(x, edge_index, edge_weight, weight, bias):
    src = edge_index[1].astype(jnp.int32)
    dst = edge_index[0].astype(jnp.int32)
    partial = _sc_segment_sum()(x, src, dst, edge_weight)
    return _tc_project(partial, weight, bias.reshape(1, D))
